# Initial kernel scaffold; baseline (speedup 1.0000x reference)
import jax
import jax.numpy as jnp
from jax.experimental import pallas as pl


def kernel(pos, vel, edge_index, mW1, mb1, mg1, mbe1, mW2, mb2, mg2, mbe2, uW1, ub1, ug1, ube1, uW2, ub2, ug2, ube2, pW, pb):
    raise NotImplementedError("write your pallas kernel here")


# speedup vs baseline: 2.3366x; 2.3366x over previous
import functools

import jax
import jax.numpy as jnp
from jax import lax
from jax.experimental import pallas as pl
from jax.experimental.pallas import tpu as pltpu
from jax.experimental.pallas import tpu_sc as plsc

_PH = lax.Precision.HIGHEST

_N = 100000
_E = 1600000
_EPS = 1e-5

_NC = 2
_NS = 16
_NW = _NC * _NS

_NPAD = 100352
_BS_PRE = 2048

_EPAD = 1638400
_EPW = _EPAD // _NW
_B1 = 1024
_NCH1 = _EPW // _B1

_BT2 = 6400
_G2 = _EPAD // _BT2

_NHALF = _N // 2
_TBL = 50016
_ZR = _TBL // _NS
_EPT = _EPAD // _NS
_B3 = 512
_NCH3 = _EPT // _B3
_RPT = _NHALF // _NS

_BN = 2000

_mesh = plsc.VectorSubcoreMesh(core_axis_name="c", subcore_axis_name="s")


@functools.partial(
    pl.kernel,
    out_type=(jax.ShapeDtypeStruct((_EPAD, 32), jnp.float32),
              jax.ShapeDtypeStruct((_NW, 64), jnp.float32)),
    mesh=_mesh,
    compiler_params=pltpu.CompilerParams(use_tc_tiling_on_sc=False),
    scratch_types=[
        pltpu.VMEM((_B1 // 128, 128), jnp.int32),
        pltpu.VMEM((_B1 // 128, 128), jnp.int32),
        pltpu.VMEM((_B1, 32), jnp.float32),
        pltpu.VMEM((_B1, 32), jnp.float32),
        pltpu.VMEM((64,), jnp.float32),
        pltpu.SemaphoreType.DMA,
        pltpu.SemaphoreType.DMA,
    ],
)
def _sc_edge(p1, p2, dsti, srci, z1_out, stats_out,
             dstv, srcv, r1, r2, statv, sem1, sem2):
    c = lax.axis_index("c")
    s = lax.axis_index("s")
    wid = s * _NC + c
    zero = jnp.zeros((16,), jnp.float32)

    def chunk(ci, carry):
        ebase = wid * _EPW + ci * _B1
        ibase = wid * (_EPW // 128) + ci * (_B1 // 128)
        pltpu.sync_copy(dsti.at[pl.ds(ibase, _B1 // 128)], dstv)
        pltpu.sync_copy(srci.at[pl.ds(ibase, _B1 // 128)], srcv)
        cps = []
        for g in range(_B1 // 128):
            cps.append(pltpu.async_copy(
                p1.at[dstv.at[g]], r1.at[pl.ds(g * 128, 128)], sem1))
            cps.append(pltpu.async_copy(
                p2.at[srcv.at[g]], r2.at[pl.ds(g * 128, 128)], sem2))
        for cp in cps:
            cp.wait()

        def rows(rb, cy):
            s0, s1v, q0, q1 = cy
            for k in range(8):
                i = rb * 8 + k
                a0 = r1[i, pl.ds(0, 16)] + r2[i, pl.ds(0, 16)]
                a1 = r1[i, pl.ds(16, 16)] + r2[i, pl.ds(16, 16)]
                r1[i, pl.ds(0, 16)] = a0
                r1[i, pl.ds(16, 16)] = a1
                s0 = s0 + a0
                s1v = s1v + a1
                q0 = q0 + a0 * a0
                q1 = q1 + a1 * a1
            return (s0, s1v, q0, q1)

        carry = lax.fori_loop(0, _B1 // 8, rows, carry)
        pltpu.sync_copy(r1, z1_out.at[pl.ds(ebase, _B1)])
        return carry

    s0, s1v, q0, q1 = lax.fori_loop(0, _NCH1, chunk, (zero, zero, zero, zero))
    statv[pl.ds(0, 16)] = s0
    statv[pl.ds(16, 16)] = s1v
    statv[pl.ds(32, 16)] = q0
    statv[pl.ds(48, 16)] = q1
    pltpu.sync_copy(statv, stats_out.at[wid])


@functools.partial(
    pl.kernel,
    out_type=jax.ShapeDtypeStruct((_N, 32), jnp.float32),
    mesh=_mesh,
    compiler_params=pltpu.CompilerParams(use_tc_tiling_on_sc=False,
                                         internal_scratch_in_bytes=65536),
    scratch_types=[
        pltpu.VMEM((_B3 // 128, 128), jnp.int32),
        pltpu.VMEM((_B3, 32), jnp.float32),
        pltpu.VMEM((64,), jnp.float32),
        pltpu.VMEM_SHARED((_TBL, 32), jnp.float32),
    ],
)
def _sc_scatter(z2, dsti, st, aggr_out, dstv, rows, stv, shared):
    c = lax.axis_index("c")
    s = lax.axis_index("s")

    def zrow(rb, _):
        for k in range(8):
            i = rb * 8 + k
            rows[i, pl.ds(0, 16)] = jnp.zeros((16,), jnp.float32)
            rows[i, pl.ds(16, 16)] = jnp.zeros((16,), jnp.float32)
        return 0

    lax.fori_loop(0, _B3 // 8, zrow, 0)
    for q in range(_ZR // _B3):
        pltpu.sync_copy(rows, shared.at[pl.ds(s * _ZR + q * _B3, _B3)])
    rem = _ZR % _B3
    if rem:
        pltpu.sync_copy(rows.at[pl.ds(0, rem)],
                        shared.at[pl.ds(s * _ZR + (_ZR // _B3) * _B3, rem)])
    pltpu.sync_copy(st, stv)
    plsc.subcore_barrier()
    sA = stv[pl.ds(0, 16)]
    sB = stv[pl.ds(16, 16)]
    tA = stv[pl.ds(32, 16)]
    tB = stv[pl.ds(48, 16)]
    cbase = c * _NHALF

    def chunk(ci, _):
        ebase = s * _EPT + ci * _B3
        ibase = s * (_EPT // 128) + ci * (_B3 // 128)
        pltpu.sync_copy(dsti.at[pl.ds(ibase, _B3 // 128)], dstv)
        pltpu.sync_copy(z2.at[pl.ds(ebase, _B3)], rows)

        def rowloop(rb, _2):
            for k in range(8):
                i = rb * 8 + k
                v0 = jnp.maximum(rows[i, pl.ds(0, 16)] * sA + tA, 0.0)
                v1 = jnp.maximum(rows[i, pl.ds(16, 16)] * sB + tB, 0.0)
                rows[i, pl.ds(0, 16)] = v0
                rows[i, pl.ds(16, 16)] = v1
            return 0

        lax.fori_loop(0, _B3 // 8, rowloop, 0)

        def idxloop(g, _2):
            for k in range(8):
                v = dstv[g, pl.ds(k * 16, 16)] - cbase
                ok = (v >= 0) & (v < _NHALF)
                v = jnp.where(ok, v, _NHALF)
                dstv[g, pl.ds(k * 16, 16)] = v
            return 0

        lax.fori_loop(0, _B3 // 128, idxloop, 0)
        for g in range(_B3 // 128):
            pltpu.sync_copy(rows.at[pl.ds(g * 128, 128)],
                            shared.at[dstv.at[g]], add=True)
        return 0

    lax.fori_loop(0, _NCH3, chunk, 0)
    plsc.subcore_barrier()
    pltpu.sync_copy(shared.at[pl.ds(s * _RPT, _RPT)],
                    aggr_out.at[pl.ds(cbase + s * _RPT, _RPT)])


def _tc_pre_body(pos_ref, vel_ref, w_ref, p1_ref, p2_ref):
    h = jnp.concatenate([pos_ref[...], vel_ref[...]], axis=1)
    p1_ref[...] = jnp.dot(h, w_ref[0:32, :], preferred_element_type=jnp.float32, precision=_PH)
    p2_ref[...] = jnp.dot(h, w_ref[32:64, :], preferred_element_type=jnp.float32, precision=_PH)


def _tc_pre(pos_p, vel_p, mW1):
    nb = _NPAD // _BS_PRE
    return pl.pallas_call(
        _tc_pre_body,
        grid=(nb,),
        in_specs=[
            pl.BlockSpec((_BS_PRE, 16), lambda i: (i, 0)),
            pl.BlockSpec((_BS_PRE, 16), lambda i: (i, 0)),
            pl.BlockSpec((64, 32), lambda i: (0, 0)),
        ],
        out_specs=[
            pl.BlockSpec((_BS_PRE, 32), lambda i: (i, 0)),
            pl.BlockSpec((_BS_PRE, 32), lambda i: (i, 0)),
        ],
        out_shape=[
            jax.ShapeDtypeStruct((_NPAD, 32), jnp.float32),
            jax.ShapeDtypeStruct((_NPAD, 32), jnp.float32),
        ],
    )(pos_p, vel_p, mW1)


def _tc_mid_body(z1_ref, w_ref, s_ref, t_ref, z2_ref, ga_ref, sa_ref):
    i = pl.program_id(0)
    a = jnp.maximum(z1_ref[...] * s_ref[0:1, :] + t_ref[0:1, :], 0.0)
    z2_ref[...] = jnp.dot(a, w_ref[...], preferred_element_type=jnp.float32, precision=_PH)
    ga = lax.dot_general(a, a, (((0,), (0,)), ((), ())),
                         preferred_element_type=jnp.float32,
                         precision=_PH)
    sa = jnp.sum(a, axis=0, keepdims=True)

    @pl.when(i == 0)
    def _():
        ga_ref[...] = jnp.zeros_like(ga_ref)
        sa_ref[...] = jnp.zeros_like(sa_ref)

    ga_ref[...] += ga
    sa_ref[0:1, :] += sa


def _tc_mid(z1, mW2, s1b, t1b):
    return pl.pallas_call(
        _tc_mid_body,
        grid=(_G2,),
        in_specs=[
            pl.BlockSpec((_BT2, 32), lambda i: (i, 0)),
            pl.BlockSpec((32, 32), lambda i: (0, 0)),
            pl.BlockSpec((8, 32), lambda i: (0, 0)),
            pl.BlockSpec((8, 32), lambda i: (0, 0)),
        ],
        out_specs=[
            pl.BlockSpec((_BT2, 32), lambda i: (i, 0)),
            pl.BlockSpec((32, 32), lambda i: (0, 0)),
            pl.BlockSpec((8, 32), lambda i: (0, 0)),
        ],
        out_shape=[
            jax.ShapeDtypeStruct((_EPAD, 32), jnp.float32),
            jax.ShapeDtypeStruct((32, 32), jnp.float32),
            jax.ShapeDtypeStruct((8, 32), jnp.float32),
        ],
    )(z1, mW2, s1b, t1b)


def _tc_node1_body(pos_ref, vel_ref, ag_ref, w_ref, z_ref, gx_ref, sx_ref):
    i = pl.program_id(0)
    x = jnp.concatenate([pos_ref[...], vel_ref[...], ag_ref[...]], axis=1)
    z_ref[...] = jnp.dot(x, w_ref[...], preferred_element_type=jnp.float32, precision=_PH)
    gx = lax.dot_general(x, x, (((0,), (0,)), ((), ())),
                         preferred_element_type=jnp.float32, precision=_PH)
    sx = jnp.sum(x, axis=0, keepdims=True)

    @pl.when(i == 0)
    def _():
        gx_ref[...] = jnp.zeros_like(gx_ref)
        sx_ref[...] = jnp.zeros_like(sx_ref)

    gx_ref[...] += gx
    sx_ref[0:1, :] += sx


def _tc_node1(pos, vel, aggr, uW1):
    nb = _N // _BN
    return pl.pallas_call(
        _tc_node1_body,
        grid=(nb,),
        in_specs=[
            pl.BlockSpec((_BN, 16), lambda i: (i, 0)),
            pl.BlockSpec((_BN, 16), lambda i: (i, 0)),
            pl.BlockSpec((_BN, 32), lambda i: (i, 0)),
            pl.BlockSpec((64, 32), lambda i: (0, 0)),
        ],
        out_specs=[
            pl.BlockSpec((_BN, 32), lambda i: (i, 0)),
            pl.BlockSpec((64, 64), lambda i: (0, 0)),
            pl.BlockSpec((8, 64), lambda i: (0, 0)),
        ],
        out_shape=[
            jax.ShapeDtypeStruct((_N, 32), jnp.float32),
            jax.ShapeDtypeStruct((64, 64), jnp.float32),
            jax.ShapeDtypeStruct((8, 64), jnp.float32),
        ],
    )(pos, vel, aggr, uW1)


def _tc_node2(zu1, uW2, sb, tb):
    return pl.pallas_call(
        _tc_mid_body,
        grid=(_N // _BN,),
        in_specs=[
            pl.BlockSpec((_BN, 32), lambda i: (i, 0)),
            pl.BlockSpec((32, 32), lambda i: (0, 0)),
            pl.BlockSpec((8, 32), lambda i: (0, 0)),
            pl.BlockSpec((8, 32), lambda i: (0, 0)),
        ],
        out_specs=[
            pl.BlockSpec((_BN, 32), lambda i: (i, 0)),
            pl.BlockSpec((32, 32), lambda i: (0, 0)),
            pl.BlockSpec((8, 32), lambda i: (0, 0)),
        ],
        out_shape=[
            jax.ShapeDtypeStruct((_N, 32), jnp.float32),
            jax.ShapeDtypeStruct((32, 32), jnp.float32),
            jax.ShapeDtypeStruct((8, 32), jnp.float32),
        ],
    )(zu1, uW2, sb, tb)


def _tc_out_body(z_ref, w_ref, b_ref, s_ref, t_ref, o_ref):
    m = jnp.maximum(z_ref[...] * s_ref[0:1, :] + t_ref[0:1, :], 0.0)
    o_ref[...] = (jnp.dot(m, w_ref[...], preferred_element_type=jnp.float32, precision=_PH)
                  + b_ref[0:1, :])


def _tc_out(zu2, pW8, pb8, sb, tb):
    return pl.pallas_call(
        _tc_out_body,
        grid=(_N // _BN,),
        in_specs=[
            pl.BlockSpec((_BN, 32), lambda i: (i, 0)),
            pl.BlockSpec((32, 8), lambda i: (0, 0)),
            pl.BlockSpec((8, 8), lambda i: (0, 0)),
            pl.BlockSpec((8, 32), lambda i: (0, 0)),
            pl.BlockSpec((8, 32), lambda i: (0, 0)),
        ],
        out_specs=pl.BlockSpec((_BN, 8), lambda i: (i, 0)),
        out_shape=jax.ShapeDtypeStruct((_N, 8), jnp.float32),
    )(zu2, pW8, pb8, sb, tb)


def _bcast8(v):
    return jnp.broadcast_to(v[None, :], (8, v.shape[0]))


def kernel(pos, vel, edge_index, mW1, mb1, mg1, mbe1, mW2, mb2, mg2, mbe2,
           uW1, ub1, ug1, ube1, uW2, ub2, ug2, ube2, pW, pb):
    f32 = jnp.float32
    pos = pos.astype(f32)
    vel = vel.astype(f32)

    pos_p = jnp.pad(pos, ((0, _NPAD - _N), (0, 0)))
    vel_p = jnp.pad(vel, ((0, _NPAD - _N), (0, 0)))
    dst = edge_index[1].astype(jnp.int32)
    src = edge_index[0].astype(jnp.int32)
    pad_idx = jnp.full((_EPAD - _E,), _N, jnp.int32)
    dst2 = jnp.concatenate([dst, pad_idx]).reshape(_EPAD // 128, 128)
    src2 = jnp.concatenate([src, pad_idx]).reshape(_EPAD // 128, 128)

    p1, p2 = _tc_pre(pos_p, vel_p, mW1)

    z1, stats = _sc_edge(p1, p2, dst2, src2)

    ssum = jnp.sum(stats, axis=0)
    m1 = ssum[0:32] / _E
    v1 = ssum[32:64] / _E - m1 * m1
    s1v = mg1 / jnp.sqrt(v1 + _EPS)
    t1v = mbe1 - s1v * m1

    z2, ga, sa = _tc_mid(z1, mW2, _bcast8(s1v), _bcast8(t1v))

    npad = _EPAD - _E
    apad = jnp.maximum(t1v, 0.0)
    sa0 = sa[0] - npad * apad
    Ga = ga - npad * jnp.outer(apad, apad)
    ma = sa0 / _E
    m2 = ma @ mW2
    cov = Ga / _E - jnp.outer(ma, ma)
    v2 = jnp.einsum('ij,ik,kj->j', mW2, cov, mW2)
    s2v = mg2 / jnp.sqrt(v2 + _EPS)
    t2v = mbe2 - s2v * m2

    st64 = jnp.concatenate([s2v, t2v])
    aggr = _sc_scatter(z2, dst2, st64)

    zu1, gx, sx = _tc_node1(pos, vel, aggr, uW1)
    mx = sx[0] / _N
    mu1 = mx @ uW1
    covx = gx / _N - jnp.outer(mx, mx)
    vu1 = jnp.einsum('ij,ik,kj->j', uW1, covx, uW1)
    su1 = ug1 / jnp.sqrt(vu1 + _EPS)
    tu1 = ube1 - su1 * mu1

    zu2, gau, sau = _tc_node2(zu1, uW2, _bcast8(su1), _bcast8(tu1))
    mau = sau[0] / _N
    mu2 = mau @ uW2
    covau = gau / _N - jnp.outer(mau, mau)
    vu2 = jnp.einsum('ij,ik,kj->j', uW2, covau, uW2)
    su2 = ug2 / jnp.sqrt(vu2 + _EPS)
    tu2 = ube2 - su2 * mu2

    pW8 = jnp.pad(pW, ((0, 0), (0, 6)))
    pb8 = jnp.broadcast_to(jnp.pad(pb, (0, 6))[None, :], (8, 8))
    out8 = _tc_out(zu2, pW8, pb8, _bcast8(su2), _bcast8(tu2))
    return out8[:, 0:2]
